# Initial kernel scaffold; baseline (speedup 1.0000x reference)
import jax
import jax.numpy as jnp
from jax.experimental import pallas as pl


def kernel(x, g_prev, batch_idx, gate_w, gate_b, nn_w, nn_b, lin_w, lin_b):
    raise NotImplementedError("write your pallas kernel here")


# speedup vs baseline: 2.8439x; 2.8439x over previous
import functools

import jax
import jax.numpy as jnp
from jax import lax
from jax.experimental import pallas as pl
from jax.experimental.pallas import tpu as pltpu
from jax.experimental.pallas import tpu_sc as plsc

N = 50000
D = 512
G = 512
AW = D + 128
NW = 32
BLK = 16
SBLK = 32
SB_PER_W = 49
BLKS_PER_W = SB_PER_W * 2
NPAD = NW * SB_PER_W * SBLK

R1 = 1000
GRID1 = N // R1


def _gate_kernel(x_ref, gw_ref, gate_ref, m_ref):
    i = pl.program_id(0)
    xb = x_ref[...]
    gb = jnp.sum(xb * gw_ref[...], axis=1)
    gate_ref[0, 0, :] = gb
    bm = jnp.max(gb)

    @pl.when(i == 0)
    def _():
        m_ref[0] = bm

    m_ref[0] = jnp.maximum(m_ref[0], bm)


def _gate(x, gate_w):
    return pl.pallas_call(
        _gate_kernel,
        grid=(GRID1,),
        in_specs=[
            pl.BlockSpec((R1, D), lambda i: (i, 0)),
            pl.BlockSpec((1, D), lambda i: (0, 0)),
        ],
        out_specs=[
            pl.BlockSpec((1, 1, R1), lambda i: (i, 0, 0)),
            pl.BlockSpec(memory_space=pltpu.SMEM),
        ],
        out_shape=[
            jax.ShapeDtypeStruct((GRID1, 1, R1), jnp.float32),
            jax.ShapeDtypeStruct((1,), jnp.float32),
        ],
    )(x, gate_w)


_SC_MESH = plsc.VectorSubcoreMesh(core_axis_name="c", subcore_axis_name="s")

SEG_PER_W = G // NW
CHUNK = NPAD // 8
CBLK = CHUNK // SBLK
NMAIN = (N // SBLK) * SBLK
TAILB = N - NMAIN


@functools.partial(
    pl.kernel,
    out_type=jax.ShapeDtypeStruct((NW, SEG_PER_W, AW), jnp.float32),
    mesh=_SC_MESH,
    scratch_types=[
        pltpu.VMEM((SBLK, D), jnp.float32),
        pltpu.VMEM((CHUNK,), jnp.float32),
        pltpu.VMEM((CHUNK,), jnp.int32),
        pltpu.VMEM((16,), jnp.float32),
        pltpu.VMEM((SEG_PER_W, AW), jnp.float32),
        pltpu.SemaphoreType.DMA,
    ],
)
def _pool(x_hbm, gate_hbm, bidx_hbm, m_hbm, out_hbm,
          xbuf, gbuf, bbuf, mbuf, acc, sem):
    c = lax.axis_index("c")
    s = lax.axis_index("s")
    w = c * 16 + s
    seg0 = w * SEG_PER_W
    seg1 = seg0 + SEG_PER_W
    zero16 = jnp.zeros((16,), jnp.float32)
    one16 = jnp.full((16,), 1, jnp.int32)
    z16i = jnp.zeros((16,), jnp.int32)
    lane0 = jnp.where(lax.iota(jnp.int32, 16) == 0, 1.0, 0.0)

    def _zrow(r, carry):
        for j in range(AW // 16):
            acc[r, pl.ds(j * 16, 16)] = zero16
        return carry

    lax.fori_loop(0, SEG_PER_W, _zrow, 0)

    pltpu.sync_copy(m_hbm, mbuf)
    m = mbuf[...][0]

    def _scan_chunk(ch, carry):
        lo_a, hi_a = carry
        pltpu.sync_copy(bidx_hbm.at[pl.ds(ch * CHUNK, CHUNK)], bbuf)

        def _scan(i, carry2):
            lo_v, hi_v = carry2
            v = bbuf[pl.ds(i * 16, 16)]
            lo_v = lo_v + jnp.where(v < seg0, one16, z16i)
            hi_v = hi_v + jnp.where(v < seg1, one16, z16i)
            return lo_v, hi_v

        return lax.fori_loop(0, CHUNK // 16, _scan, (lo_a, hi_a))

    lo_v, hi_v = lax.fori_loop(0, NPAD // CHUNK, _scan_chunk, (z16i, z16i))
    lo = lo_v[0]
    hi = hi_v[0]
    for r in range(1, 16):
        lo = lo + lo_v[r]
        hi = hi + hi_v[r]

    k0 = lo // SBLK
    k1 = jnp.minimum((hi + SBLK - 1) // SBLK, NMAIN // SBLK)

    def _do_rows(nrows, roff_in_chunk, rbase):
        for h in range(nrows // BLK):
            gv = gbuf[pl.ds(roff_in_chunk + h * BLK, BLK)]
            bv = bbuf[pl.ds(roff_in_chunk + h * BLK, BLK)]
            owned = jnp.logical_and(bv >= seg0, bv < seg1)
            ev = jnp.where(owned, jnp.exp(gv - m), 0.0)
            slv = jnp.clip(bv - seg0, 0, SEG_PER_W - 1)
            for r in range(BLK):
                rr = h * BLK + r
                er = ev[r]
                sl = slv[r]
                for j in range(D // 16):
                    plsc.addupdate(acc.at[sl, pl.ds(j * 16, 16)],
                                   xbuf[rr, pl.ds(j * 16, 16)] * er)
                plsc.addupdate(acc.at[sl, pl.ds(D, 16)], lane0 * er)

    def _chunk_body(ch, carry):
        cb0 = ch * CBLK
        kstart = jnp.clip(k0 - cb0, 0, CBLK)
        kend = jnp.clip(k1 - cb0, 0, CBLK)

        @pl.when(kend > kstart)
        def _():
            pltpu.sync_copy(gate_hbm.at[pl.ds(ch * CHUNK, CHUNK)], gbuf)
            pltpu.sync_copy(bidx_hbm.at[pl.ds(ch * CHUNK, CHUNK)], bbuf)

            def _blk(k, carry2):
                rbase = (cb0 + k) * SBLK
                pltpu.sync_copy(x_hbm.at[pl.ds(rbase, SBLK)], xbuf)
                _do_rows(SBLK, k * SBLK, rbase)
                return carry2

            lax.fori_loop(kstart, kend, _blk, 0)
        return carry

    lax.fori_loop(0, NPAD // CHUNK, _chunk_body, 0)

    @pl.when(jnp.logical_and(hi > NMAIN, lo < N))
    def _():
        pltpu.sync_copy(gate_hbm.at[pl.ds(NMAIN, TAILB)], gbuf.at[pl.ds(0, TAILB)])
        pltpu.sync_copy(bidx_hbm.at[pl.ds(NMAIN, TAILB)], bbuf.at[pl.ds(0, TAILB)])
        pltpu.sync_copy(x_hbm.at[pl.ds(NMAIN, TAILB)], xbuf.at[pl.ds(0, TAILB)])
        _do_rows(TAILB, 0, NMAIN)

    pltpu.sync_copy(acc, out_hbm.at[w])


def _mlp_kernel(acc_ref, gp_ref, nnw_ref, nnb_ref, lw1_ref, lw2_ref, lb_ref, o_ref):
    wsum = acc_ref[:, :D]
    den = jnp.sum(acc_ref[:, D:], axis=1, keepdims=True)
    inv = 1.0 / (den + 1e-16)
    sfrac = den * inv
    nt = (((1,), (1,)), ((), ()))
    gbar = lax.dot_general(wsum * inv, nnw_ref[...], nt,
                           preferred_element_type=jnp.float32)
    gbar = gbar + sfrac * nnb_ref[...]
    gp = gp_ref[...]
    h = lax.dot_general(gbar, lw1_ref[...], nt, preferred_element_type=jnp.float32)
    h = h + lax.dot_general(gp, lw2_ref[...], nt, preferred_element_type=jnp.float32)
    h = h + lb_ref[...]
    o_ref[...] = gp + jnp.maximum(h, 0.0)


def _mlp(acc, g_prev, nn_w, nn_b2, lw1, lw2, lb2):
    return pl.pallas_call(
        _mlp_kernel,
        out_shape=jax.ShapeDtypeStruct((G, D), jnp.float32),
    )(acc, g_prev, nn_w, nn_b2, lw1, lw2, lb2)


def kernel(x, g_prev, batch_idx, gate_w, gate_b, nn_w, nn_b, lin_w, lin_b):
    del gate_b
    gate3, mmax = _gate(x, gate_w)
    gate = gate3.reshape(N)
    pad = NPAD - N
    gate_p = jnp.concatenate(
        [gate, jnp.full((pad,), -1e30, jnp.float32)])
    bidx_p = jnp.concatenate(
        [batch_idx.astype(jnp.int32), jnp.full((pad,), G, jnp.int32)])
    mvec = jnp.broadcast_to(mmax, (16,))
    acc = _pool(x, gate_p.reshape(NPAD), bidx_p.reshape(NPAD), mvec)
    return _mlp(acc.reshape(G, AW), g_prev, nn_w, nn_b.reshape(1, D),
                lin_w[:, :D], lin_w[:, D:], lin_b.reshape(1, D))


# speedup vs baseline: 4.6271x; 1.6270x over previous
import functools

import jax
import jax.numpy as jnp
from jax import lax
from jax.experimental import pallas as pl
from jax.experimental.pallas import tpu as pltpu
from jax.experimental.pallas import tpu_sc as plsc

N = 50000
D = 512
G = 512
AW = D + 128
NW = 32
BLK = 16
SBLK = 32
SB_PER_W = 49
BLKS_PER_W = SB_PER_W * 2
NPAD = NW * SB_PER_W * SBLK

R1 = 1000
GRID1 = N // R1


def _gate_kernel(x_ref, gw_ref, gate_ref, m_ref):
    i = pl.program_id(0)
    xb = x_ref[...]
    gb = jnp.sum(xb * gw_ref[...], axis=1)
    gate_ref[0, 0, :] = gb
    bm = jnp.max(gb)

    @pl.when(i == 0)
    def _():
        m_ref[0] = bm

    m_ref[0] = jnp.maximum(m_ref[0], bm)


def _gate(x, gate_w):
    return pl.pallas_call(
        _gate_kernel,
        grid=(GRID1,),
        in_specs=[
            pl.BlockSpec((R1, D), lambda i: (i, 0)),
            pl.BlockSpec((1, D), lambda i: (0, 0)),
        ],
        out_specs=[
            pl.BlockSpec((1, 1, R1), lambda i: (i, 0, 0)),
            pl.BlockSpec(memory_space=pltpu.SMEM),
        ],
        out_shape=[
            jax.ShapeDtypeStruct((GRID1, 1, R1), jnp.float32),
            jax.ShapeDtypeStruct((1,), jnp.float32),
        ],
    )(x, gate_w)


_SC_MESH = plsc.VectorSubcoreMesh(core_axis_name="c", subcore_axis_name="s")

SEG_PER_W = G // NW
CHUNK = NPAD // 8
CBLK = CHUNK // SBLK
NMAIN = (N // SBLK) * SBLK
TAILB = N - NMAIN


@functools.partial(
    pl.kernel,
    out_type=jax.ShapeDtypeStruct((NW, SEG_PER_W * AW), jnp.float32),
    mesh=_SC_MESH,
    scratch_types=[
        pltpu.VMEM((2 * SBLK * D,), jnp.float32),
        pltpu.VMEM((CHUNK,), jnp.float32),
        pltpu.VMEM((CHUNK,), jnp.int32),
        pltpu.VMEM((16,), jnp.float32),
        pltpu.VMEM((SEG_PER_W * AW,), jnp.float32),
        pltpu.SemaphoreType.DMA,
    ],
)
def _pool(x_flat, gate_hbm, bidx_hbm, m_hbm, out_hbm,
          xbuf, gbuf, bbuf, mbuf, acc, sem):
    c = lax.axis_index("c")
    s = lax.axis_index("s")
    w = c * 16 + s
    seg0 = w * SEG_PER_W
    seg1 = seg0 + SEG_PER_W
    zero16 = jnp.zeros((16,), jnp.float32)
    one16f = jnp.ones((16,), jnp.float32)
    one16 = jnp.full((16,), 1, jnp.int32)
    z16i = jnp.zeros((16,), jnp.int32)
    lane0 = jnp.where(lax.iota(jnp.int32, 16) == 0, 1.0, 0.0)

    def _zrow(r, carry):
        for j in range(AW // 16):
            acc[pl.ds(r * AW + j * 16, 16)] = zero16
        return carry

    lax.fori_loop(0, SEG_PER_W, _zrow, 0)

    pltpu.sync_copy(m_hbm, mbuf)
    m = mbuf[...][0]

    def _scan_chunk(ch, carry):
        lo_a, hi_a = carry
        pltpu.sync_copy(bidx_hbm.at[pl.ds(ch * CHUNK, CHUNK)], bbuf)

        def _scan(i, carry2):
            lo_v, hi_v = carry2
            v = bbuf[pl.ds(i * 16, 16)]
            lo_v = lo_v + jnp.where(v < seg0, one16, z16i)
            hi_v = hi_v + jnp.where(v < seg1, one16, z16i)
            return lo_v, hi_v

        return lax.fori_loop(0, CHUNK // 16, _scan, (lo_a, hi_a))

    lo_v, hi_v = lax.fori_loop(0, NPAD // CHUNK, _scan_chunk, (z16i, z16i))
    lo = lo_v[0]
    hi = hi_v[0]
    for r in range(1, 16):
        lo = lo + lo_v[r]
        hi = hi + hi_v[r]

    k0 = lo // SBLK
    k1 = jnp.minimum((hi + SBLK - 1) // SBLK, NMAIN // SBLK)

    NREG = D // 16

    XB = SBLK * D

    _splat_idx = [jnp.full((16,), r, jnp.int32) for r in range(BLK)]

    def _row(xrow, erv, sl, st):
        cur, den, regs = st
        is_new = sl != cur

        @pl.when(is_new)
        def _():
            for j in range(NREG):
                acc[pl.ds(cur * AW + j * 16, 16)] = regs[j]
            acc[pl.ds(cur * AW + D, 16)] = den * lane0

        keep = 1.0 - is_new.astype(jnp.float32)
        new_regs = tuple(
            regs[j] * keep + xbuf[pl.ds(xrow + j * 16, 16)] * erv
            for j in range(NREG))
        return (sl, den * keep + erv * lane0, new_regs)

    def _do16(xbase, goff, st):
        gv = gbuf[pl.ds(goff, BLK)]
        bv = bbuf[pl.ds(goff, BLK)]
        owned = jnp.logical_and(bv >= seg0, bv < seg1)
        ev = jnp.where(owned, jnp.exp(gv - m), 0.0)
        slv = jnp.clip(bv - seg0, 0, SEG_PER_W - 1)
        for r in range(BLK):
            erv = ev.at[_splat_idx[r]].get(mode="promise_in_bounds")
            st = _row(xbase + r * D, erv, slv[r], st)
        return st

    def _chunk_body(ch, st):
        cb0 = ch * CBLK
        kstart = jnp.clip(k0 - cb0, 0, CBLK)
        kend = jnp.clip(k1 - cb0, 0, CBLK)

        @pl.when(kend > kstart)
        def _():
            pltpu.sync_copy(gate_hbm.at[pl.ds(ch * CHUNK, CHUNK)], gbuf)
            pltpu.sync_copy(bidx_hbm.at[pl.ds(ch * CHUNK, CHUNK)], bbuf)
            slot0 = (kstart % 2) * XB
            pltpu.async_copy(
                x_flat.at[pl.ds((cb0 + kstart) * SBLK * D, XB)],
                xbuf.at[pl.ds(slot0, XB)], sem)

        def _blk(k, st2):
            slot = (k % 2) * XB
            pltpu.make_async_copy(
                x_flat.at[pl.ds(0, XB)],
                xbuf.at[pl.ds(slot, XB)], sem).wait()

            @pl.when(k + 1 < kend)
            def _():
                nslot = ((k + 1) % 2) * XB
                pltpu.async_copy(
                    x_flat.at[pl.ds((cb0 + k + 1) * SBLK * D, XB)],
                    xbuf.at[pl.ds(nslot, XB)], sem)

            for h in range(SBLK // BLK):
                st2 = _do16(slot + h * BLK * D, k * SBLK + h * BLK, st2)
            return st2

        return lax.fori_loop(kstart, kend, _blk, st)

    st = (jnp.int32(0), zero16,
          tuple(zero16 for _ in range(NREG)))
    st = lax.fori_loop(0, NPAD // CHUNK, _chunk_body, st)

    pltpu.sync_copy(gate_hbm.at[pl.ds(NMAIN, TAILB)],
                    gbuf.at[pl.ds(0, TAILB)])
    pltpu.sync_copy(bidx_hbm.at[pl.ds(NMAIN, TAILB)],
                    bbuf.at[pl.ds(0, TAILB)])
    pltpu.sync_copy(x_flat.at[pl.ds(NMAIN * D, TAILB * D)],
                    xbuf.at[pl.ds(0, TAILB * D)])
    st = _do16(0, 0, st)

    cur, den, regs = st
    for j in range(NREG):
        acc[pl.ds(cur * AW + j * 16, 16)] = regs[j]
    acc[pl.ds(cur * AW + D, 16)] = den * lane0

    pltpu.sync_copy(acc, out_hbm.at[w])


def _mlp_kernel(acc_ref, gp_ref, nnw_ref, nnb_ref, lw1_ref, lw2_ref, lb_ref, o_ref):
    wsum = acc_ref[:, :D]
    den = jnp.sum(acc_ref[:, D:], axis=1, keepdims=True)
    inv = 1.0 / (den + 1e-16)
    sfrac = den * inv
    nt = (((1,), (1,)), ((), ()))
    gbar = lax.dot_general(wsum * inv, nnw_ref[...], nt,
                           preferred_element_type=jnp.float32)
    gbar = gbar + sfrac * nnb_ref[...]
    gp = gp_ref[...]
    h = lax.dot_general(gbar, lw1_ref[...], nt, preferred_element_type=jnp.float32)
    h = h + lax.dot_general(gp, lw2_ref[...], nt, preferred_element_type=jnp.float32)
    h = h + lb_ref[...]
    o_ref[...] = gp + jnp.maximum(h, 0.0)


def _mlp(acc, g_prev, nn_w, nn_b2, lw1, lw2, lb2):
    return pl.pallas_call(
        _mlp_kernel,
        out_shape=jax.ShapeDtypeStruct((G, D), jnp.float32),
    )(acc, g_prev, nn_w, nn_b2, lw1, lw2, lb2)


def kernel(x, g_prev, batch_idx, gate_w, gate_b, nn_w, nn_b, lin_w, lin_b):
    del gate_b
    gate3, mmax = _gate(x, gate_w)
    gate = gate3.reshape(N)
    pad = NPAD - N
    gate_p = jnp.concatenate(
        [gate, jnp.full((pad,), -1e30, jnp.float32)])
    bidx_p = jnp.concatenate(
        [batch_idx.astype(jnp.int32), jnp.full((pad,), G, jnp.int32)])
    mvec = jnp.broadcast_to(mmax, (16,))
    acc = _pool(x.reshape(N * D), gate_p.reshape(NPAD),
                bidx_p.reshape(NPAD), mvec)
    return _mlp(acc.reshape(G, AW), g_prev, nn_w, nn_b.reshape(1, D),
                lin_w[:, :D], lin_w[:, D:], lin_b.reshape(1, D))
